# Initial kernel scaffold; baseline (speedup 1.0000x reference)
import functools

import jax
import jax.numpy as jnp
from jax import lax
from jax.experimental import pallas as pl
from jax.experimental.pallas import tpu as pltpu
from jax.experimental.pallas import tpu_sc as plsc

_N = 10000
_E = 320000
_H = 256
_IR = 128
_FLEN = 128
_LANES = 16
_NTILES = 16
_CL = 128
_EC = _E // _CL
_KPT = 157
_KLAST = _EC - 15 * _KPT
_RPT = _N // _NTILES


def _sc_mesh():
    return plsc.VectorSubcoreMesh(core_axis_name="c", subcore_axis_name="s")


def _make_deg(n):
    wpt = _KPT * _CL
    wlast = _KLAST * _CL

    @functools.partial(
        pl.kernel,
        out_type=jax.ShapeDtypeStruct((2, n), jnp.float32),
        mesh=_sc_mesh(),
        scratch_types=[
            pltpu.VMEM((wpt,), jnp.int32),
            pltpu.VMEM((n,), jnp.float32),
            pltpu.VMEM_SHARED((n,), jnp.float32),
        ],
    )
    def deg_k(ef, deg2, idx1, dloc, dsh):
        c = lax.axis_index("c")
        s = lax.axis_index("s")

        def zb(i, _):
            dloc[pl.ds(i * _LANES, _LANES)] = jnp.zeros((_LANES,), jnp.float32)
            return 0

        lax.fori_loop(0, n // _LANES, zb, 0)

        @pl.when(s == 0)
        def _():
            pltpu.sync_copy(dloc, dsh)

        @pl.when(s < 15)
        def _():
            pltpu.sync_copy(ef.at[c, pl.ds(s * wpt, wpt)], idx1.at[pl.ds(0, wpt)])

        @pl.when(s == 15)
        def _():
            pltpu.sync_copy(ef.at[c, pl.ds(15 * wpt, wlast)], idx1.at[pl.ds(0, wlast)])

        plsc.subcore_barrier()
        nv = jnp.where(s < 15, wpt // _LANES, wlast // _LANES)
        ones = jnp.full((_LANES,), 1.0, jnp.float32)

        def acc(i, _):
            idx = idx1[pl.ds(i * _LANES, _LANES)]
            plsc.addupdate_scatter(dloc, [idx], ones)
            return 0

        lax.fori_loop(0, nv, acc, 0)
        pltpu.sync_copy(dloc, dsh, add=True)
        plsc.subcore_barrier()

        @pl.when(s == 0)
        def _():
            pltpu.sync_copy(dsh, deg2.at[c])

    return deg_k


def _make_prop(n, f):
    wpt = _KPT * _CL
    wlast = _KLAST * _CL
    nhalf = (_KPT + 1) // 2

    @functools.partial(
        pl.kernel,
        out_type=jax.ShapeDtypeStruct((2, n, f), jnp.float32),
        mesh=_sc_mesh(),
        scratch_types=[
            pltpu.VMEM((wpt,), jnp.int32),
            pltpu.VMEM((_KPT, _CL), jnp.int32),
            pltpu.VMEM((_CL, f), jnp.float32),
            pltpu.VMEM((_CL, f), jnp.float32),
            pltpu.VMEM_SHARED((n, f), jnp.float32),
            pltpu.SemaphoreType.DMA,
            pltpu.SemaphoreType.DMA,
        ],
    )
    def prop_k(xs2, ef2, e3, zrows, agg, isrc, idst, rows0, rows1, aggsh, g0, g1):
        c = lax.axis_index("c")
        s = lax.axis_index("s")

        @pl.when(s < 15)
        def _():
            pltpu.sync_copy(ef2.at[c, pl.ds(s * wpt, wpt)], isrc.at[pl.ds(0, wpt)])
            pltpu.sync_copy(e3.at[1, pl.ds(s * _KPT, _KPT)], idst.at[pl.ds(0, _KPT)])

        @pl.when(s == 15)
        def _():
            pltpu.sync_copy(ef2.at[c, pl.ds(15 * wpt, wlast)], isrc.at[pl.ds(0, wlast)])
            pltpu.sync_copy(e3.at[1, pl.ds(15 * _KPT, _KLAST)], idst.at[pl.ds(0, _KLAST)])

        nk = jnp.where(s < 15, _KPT, _KLAST)
        pltpu.sync_copy(zrows, aggsh.at[pl.ds(s * _RPT, _RPT)])
        plsc.subcore_barrier()

        dummy = xs2.at[pl.ds(0, _CL)]

        def gstart(k, rbuf, sem):
            pltpu.async_copy(xs2.at[isrc.at[pl.ds(k * _CL, _CL)]], rbuf, sem)

        def gwait(rbuf, sem):
            pltpu.make_async_copy(dummy, rbuf, sem).wait()

        def scat(k, rbuf):
            pltpu.sync_copy(rbuf, aggsh.at[idst.at[k]], add=True)

        gstart(0, rows0, g0)

        def body(k2, _):
            k0 = k2 * 2
            k1 = k0 + 1

            @pl.when(k0 < nk)
            def _():
                gwait(rows0, g0)

                @pl.when(k1 < nk)
                def _():
                    gstart(k1, rows1, g1)

                scat(k0, rows0)

                @pl.when(k1 < nk)
                def _():
                    gwait(rows1, g1)

                    @pl.when(k1 + 1 < nk)
                    def _():
                        gstart(k1 + 1, rows0, g0)

                    scat(k1, rows1)

            return 0

        lax.fori_loop(0, nhalf, body, 0)
        plsc.subcore_barrier()
        pltpu.sync_copy(aggsh.at[pl.ds(s * _RPT, _RPT)],
                        agg.at[c, pl.ds(s * _RPT, _RPT)])

    return prop_k


def _make_prep(n):
    bn = 1000

    def body(h_ref, do_ref, di_ref, xs_ref, ns_ref, nd_ref):
        nsv = lax.rsqrt(jnp.maximum(do_ref[...], 1.0))
        ndv = lax.rsqrt(jnp.maximum(di_ref[...], 1.0))
        xsv = h_ref[...] * nsv
        xs_ref[0] = xsv[:, : _FLEN // 2]
        xs_ref[1] = xsv[:, _FLEN // 2:]
        ns_ref[...] = nsv
        nd_ref[...] = ndv

    return pl.pallas_call(
        body,
        grid=(n // bn,),
        in_specs=[
            pl.BlockSpec((bn, _FLEN), lambda i: (i, 0)),
            pl.BlockSpec((bn, 1), lambda i: (i, 0)),
            pl.BlockSpec((bn, 1), lambda i: (i, 0)),
        ],
        out_specs=[
            pl.BlockSpec((2, bn, _FLEN // 2), lambda i: (0, i, 0)),
            pl.BlockSpec((bn, 1), lambda i: (i, 0)),
            pl.BlockSpec((bn, 1), lambda i: (i, 0)),
        ],
        out_shape=[
            jax.ShapeDtypeStruct((2, n, _FLEN // 2), jnp.float32),
            jax.ShapeDtypeStruct((n, 1), jnp.float32),
            jax.ShapeDtypeStruct((n, 1), jnp.float32),
        ],
    )


def _make_tmat(n, fin, fout):
    bn = 1000

    def body(agg_ref, nd_ref, ns_ref, w_ref, b_ref, out_ref):
        x = jnp.concatenate([agg_ref[0], agg_ref[1]], axis=1) * nd_ref[...]
        y = jnp.dot(x, w_ref[...], preferred_element_type=jnp.float32) + b_ref[...]
        y = jnp.maximum(y, 0.0) * ns_ref[...]
        out_ref[0] = y[:, :fout]
        out_ref[1] = y[:, fout:]

    return pl.pallas_call(
        body,
        grid=(n // bn,),
        in_specs=[
            pl.BlockSpec((2, bn, fin), lambda i: (0, i, 0)),
            pl.BlockSpec((bn, 1), lambda i: (i, 0)),
            pl.BlockSpec((bn, 1), lambda i: (i, 0)),
            pl.BlockSpec((2 * fin, 2 * fout), lambda i: (0, 0)),
            pl.BlockSpec((1, 2 * fout), lambda i: (0, 0)),
        ],
        out_specs=pl.BlockSpec((2, bn, fout), lambda i: (0, i, 0)),
        out_shape=jax.ShapeDtypeStruct((2, n, fout), jnp.float32),
    )


def _make_head(n, f):
    bn = 1000
    feat = 2 * f

    def body(agg_ref, nd_ref, w6, b6, wl1, bl1, wl2, bl2, out_ref, acc):
        i = pl.program_id(0)

        @pl.when(i == 0)
        def _():
            acc[...] = jnp.zeros((1, feat), jnp.float32)

        x = jnp.concatenate([agg_ref[0], agg_ref[1]], axis=1) * nd_ref[...]
        acc[...] += jnp.sum(x, axis=0, keepdims=True)

        @pl.when(i == pl.num_programs(0) - 1)
        def _():
            pooled = (jnp.dot(acc[...], w6[...], preferred_element_type=jnp.float32)
                      + float(n) * b6[...])
            y1 = jnp.maximum(
                jnp.dot(pooled, wl1[...], preferred_element_type=jnp.float32)
                + bl1[...], 0.0)
            out_ref[...] = (jnp.dot(y1, wl2[...], preferred_element_type=jnp.float32)
                            + bl2[...])

    return pl.pallas_call(
        body,
        grid=(n // bn,),
        in_specs=[
            pl.BlockSpec((2, bn, f), lambda i: (0, i, 0)),
            pl.BlockSpec((bn, 1), lambda i: (i, 0)),
            pl.BlockSpec((feat, feat), lambda i: (0, 0)),
            pl.BlockSpec((1, feat), lambda i: (0, 0)),
            pl.BlockSpec((feat, feat), lambda i: (0, 0)),
            pl.BlockSpec((1, feat), lambda i: (0, 0)),
            pl.BlockSpec((feat, _IR), lambda i: (0, 0)),
            pl.BlockSpec((1, _IR), lambda i: (0, 0)),
        ],
        out_specs=pl.BlockSpec((1, _IR), lambda i: (0, 0)),
        out_shape=jax.ShapeDtypeStruct((1, _IR), jnp.float32),
        scratch_shapes=[pltpu.VMEM((1, feat), jnp.float32)],
    )


_deg = _make_deg(_N)
_prop64 = _make_prop(_N, 64)
_prop128 = _make_prop(_N, 128)
_prep = _make_prep(_N)
_tmat64 = _make_tmat(_N, 64, 128)
_tmat128 = _make_tmat(_N, 128, 128)
_head = _make_head(_N, 128)


def kernel(h, edge_index, W0, b0, W1, b1, W2, b2, W3, b3, W4, b4, W5, b5,
           W6, b6, Wl1, bl1, Wl2, bl2):
    src = edge_index[0]
    ef2 = jnp.stack([src, src + jnp.int32(_N)])
    e3 = edge_index.reshape(2, _EC, _CL)

    deg2 = _deg(edge_index)
    dego = deg2[0].reshape(_N, 1)
    degi = deg2[1].reshape(_N, 1)
    xs0, nsc, ndc = _prep(h, dego, degi)

    z64 = jnp.zeros((_RPT, 64), jnp.float32)
    z128 = jnp.zeros((_RPT, 128), jnp.float32)

    agg = _prop64(xs0.reshape(2 * _N, 64), ef2, e3, z64)
    x = _tmat64(agg, ndc, nsc, W0, b0.reshape(1, _H))
    for W, b in ((W1, b1), (W2, b2), (W3, b3), (W4, b4), (W5, b5)):
        agg = _prop128(x.reshape(2 * _N, 128), ef2, e3, z128)
        x = _tmat128(agg, ndc, nsc, W, b.reshape(1, _H))
    agg = _prop128(x.reshape(2 * _N, 128), ef2, e3, z128)
    return _head(agg, ndc, W6, b6.reshape(1, _H), Wl1, bl1.reshape(1, _H),
                 Wl2, bl2.reshape(1, _IR))


# speedup vs baseline: 2.6942x; 2.6942x over previous
import functools

import jax
import jax.numpy as jnp
from jax import lax
from jax.experimental import pallas as pl
from jax.experimental.pallas import tpu as pltpu
from jax.experimental.pallas import tpu_sc as plsc

_N = 10000
_E = 320000
_H = 256
_IR = 128
_FLEN = 128
_NTILES = 16
_CL = 128
_KPT = 160
_ECP = _NTILES * _KPT
_EP = _ECP * _CL
_NP = 10240
_RPT = _NP // _NTILES


def _sc_mesh():
    return plsc.VectorSubcoreMesh(core_axis_name="c", subcore_axis_name="s")


_IBLK = 16


def _make_prop(f, kpt=_KPT):
    @functools.partial(
        pl.kernel,
        out_type=jax.ShapeDtypeStruct((2, _NP, f), jnp.float32),
        name=f"gcn_prop_f{f}_k{kpt}",
        mesh=_sc_mesh(),
        scratch_types=[
            pltpu.VMEM((_IBLK, _CL), jnp.int32),
            pltpu.VMEM((_IBLK, _CL), jnp.int32),
            pltpu.VMEM((_CL, f), jnp.float32),
            pltpu.VMEM((_CL, f), jnp.float32),
            pltpu.VMEM_SHARED((_NP, f), jnp.float32),
            pltpu.SemaphoreType.DMA,
            pltpu.SemaphoreType.DMA,
        ],
    )
    def prop_k(xs2, srcp3, dstp3, zrows, agg, isrc, idst, rows0, rows1,
               aggsh, g0, g1):
        c = lax.axis_index("c")
        s = lax.axis_index("s")

        pltpu.sync_copy(zrows, aggsh.at[pl.ds(s * _RPT, _RPT)])
        plsc.subcore_barrier()

        dummy = xs2.at[pl.ds(0, _CL)]

        def gstart(j, rbuf, sem):
            pltpu.async_copy(xs2.at[isrc.at[j]], rbuf, sem)

        def gwait(rbuf, sem):
            pltpu.make_async_copy(dummy, rbuf, sem).wait()

        def scat(j, rbuf):
            pltpu.sync_copy(rbuf, aggsh.at[idst.at[j]], add=True)

        def blk(ib, _):
            base = s * kpt + ib * _IBLK
            pltpu.sync_copy(srcp3.at[c, pl.ds(base, _IBLK)], isrc)
            pltpu.sync_copy(dstp3.at[c, pl.ds(base, _IBLK)], idst)
            gstart(0, rows0, g0)

            def inner(j2, _):
                j0 = j2 * 2
                j1 = j0 + 1
                gwait(rows0, g0)
                gstart(j1, rows1, g1)
                scat(j0, rows0)
                gwait(rows1, g1)

                @pl.when(j1 + 1 < _IBLK)
                def _():
                    gstart(j1 + 1, rows0, g0)

                scat(j1, rows1)
                return 0

            lax.fori_loop(0, _IBLK // 2, inner, 0)
            return 0

        lax.fori_loop(0, kpt // _IBLK, blk, 0)
        plsc.subcore_barrier()
        pltpu.sync_copy(aggsh.at[pl.ds(s * _RPT, _RPT)],
                        agg.at[c, pl.ds(s * _RPT, _RPT)])

    return prop_k


def _make_prep(n):
    bn = 1000

    def body(h_ref, do_ref, di_ref, xs_ref, ns_ref, nd_ref):
        nsv = lax.rsqrt(jnp.maximum(do_ref[...], 1.0))
        ndv = lax.rsqrt(jnp.maximum(di_ref[...], 1.0))
        xs_ref[...] = h_ref[...] * nsv
        ns_ref[...] = nsv
        nd_ref[...] = ndv

    return pl.pallas_call(
        body,
        grid=(n // bn,),
        in_specs=[
            pl.BlockSpec((bn, _FLEN), lambda i: (i, 0)),
            pl.BlockSpec((bn, 1), lambda i: (i, 0)),
            pl.BlockSpec((bn, 1), lambda i: (i, 0)),
        ],
        out_specs=[
            pl.BlockSpec((bn, _FLEN), lambda i: (i, 0)),
            pl.BlockSpec((bn, 1), lambda i: (i, 0)),
            pl.BlockSpec((bn, 1), lambda i: (i, 0)),
        ],
        out_shape=[
            jax.ShapeDtypeStruct((n, _FLEN), jnp.float32),
            jax.ShapeDtypeStruct((n, 1), jnp.float32),
            jax.ShapeDtypeStruct((n, 1), jnp.float32),
        ],
    )


def _make_tmat(n, fin, fout, sum_planes=False):
    bn = 640

    def body(agg_ref, nd_ref, ns_ref, w_ref, b_ref, out_ref):
        if sum_planes:
            x = (agg_ref[0] + agg_ref[1]) * nd_ref[...]
        else:
            x = jnp.concatenate([agg_ref[0], agg_ref[1]], axis=1) * nd_ref[...]
        y = jnp.dot(x, w_ref[...], preferred_element_type=jnp.float32) + b_ref[...]
        y = jnp.maximum(y, 0.0) * ns_ref[...]
        out_ref[0] = y[:, :fout]
        out_ref[1] = y[:, fout:]

    return pl.pallas_call(
        body,
        grid=(n // bn,),
        in_specs=[
            pl.BlockSpec((2, bn, fin), lambda i: (0, i, 0)),
            pl.BlockSpec((bn, 1), lambda i: (i, 0)),
            pl.BlockSpec((bn, 1), lambda i: (i, 0)),
            pl.BlockSpec((fin if sum_planes else 2 * fin, 2 * fout),
                         lambda i: (0, 0)),
            pl.BlockSpec((1, 2 * fout), lambda i: (0, 0)),
        ],
        out_specs=pl.BlockSpec((2, bn, fout), lambda i: (0, i, 0)),
        out_shape=jax.ShapeDtypeStruct((2, n, fout), jnp.float32),
    )


def _make_head(n, f):
    bn = 640
    feat = 2 * f

    def body(agg_ref, nd_ref, w6, b6, wl1, bl1, wl2, bl2, out_ref, acc):
        i = pl.program_id(0)

        @pl.when(i == 0)
        def _():
            acc[...] = jnp.zeros((1, feat), jnp.float32)

        x = jnp.concatenate([agg_ref[0], agg_ref[1]], axis=1) * nd_ref[...]
        acc[...] += jnp.sum(x, axis=0, keepdims=True)

        @pl.when(i == pl.num_programs(0) - 1)
        def _():
            pooled = (jnp.dot(acc[...], w6[...], preferred_element_type=jnp.float32)
                      + float(_N) * b6[...])
            y1 = jnp.maximum(
                jnp.dot(pooled, wl1[...], preferred_element_type=jnp.float32)
                + bl1[...], 0.0)
            out_ref[...] = (jnp.dot(y1, wl2[...], preferred_element_type=jnp.float32)
                            + bl2[...])

    return pl.pallas_call(
        body,
        grid=(n // bn,),
        in_specs=[
            pl.BlockSpec((2, bn, f), lambda i: (0, i, 0)),
            pl.BlockSpec((bn, 1), lambda i: (i, 0)),
            pl.BlockSpec((feat, feat), lambda i: (0, 0)),
            pl.BlockSpec((1, feat), lambda i: (0, 0)),
            pl.BlockSpec((feat, feat), lambda i: (0, 0)),
            pl.BlockSpec((1, feat), lambda i: (0, 0)),
            pl.BlockSpec((feat, _IR), lambda i: (0, 0)),
            pl.BlockSpec((1, _IR), lambda i: (0, 0)),
        ],
        out_specs=pl.BlockSpec((1, _IR), lambda i: (0, 0)),
        out_shape=jax.ShapeDtypeStruct((1, _IR), jnp.float32),
        scratch_shapes=[pltpu.VMEM((1, feat), jnp.float32)],
    )


_prep = _make_prep(_N)
_tmat0 = _make_tmat(_NP, 128, 128, sum_planes=True)
_tmat128 = _make_tmat(_NP, 128, 128)
_head = _make_head(_NP, 128)

_CACHE = {}


def _get_prop(f, kpt=_KPT):
    if (f, kpt) not in _CACHE:
        _CACHE[(f, kpt)] = _make_prop(f, kpt)
    return _CACHE[(f, kpt)]


def _pad_table(x2):
    f = x2.shape[2]
    return jnp.concatenate(
        [x2.reshape(2 * _NP, f), jnp.zeros((8, f), x2.dtype)], axis=0)


def kernel(h, edge_index, W0, b0, W1, b1, W2, b2, W3, b3, W4, b4, W5, b5,
           W6, b6, Wl1, bl1, Wl2, bl2):
    src = edge_index[0]
    dst = edge_index[1]
    npad = _EP - _E
    padg = jnp.full((npad,), 2 * _NP, jnp.int32)
    pads = jnp.full((npad,), _N, jnp.int32)
    srcp = jnp.concatenate([src, pads])
    dstp = jnp.concatenate([dst, pads])
    srcp3 = jnp.stack([jnp.concatenate([src, padg]),
                       jnp.concatenate([src + jnp.int32(_NP), padg])
                       ]).reshape(2, _ECP, _CL)
    dst3 = jnp.stack([dstp, dstp]).reshape(2, _ECP, _CL)
    deg3 = jnp.stack([srcp, dstp]).reshape(2, _ECP, _CL)

    padg0 = jnp.full((npad,), _NP, jnp.int32)
    srcp3_l0 = jnp.concatenate([src, padg0]).reshape(2, _ECP // 2, _CL)
    dst3_l0 = jnp.concatenate([dst, pads]).reshape(2, _ECP // 2, _CL)

    z128 = jnp.zeros((_RPT, 128), jnp.float32)
    ones_tab = jnp.concatenate(
        [jnp.ones((2 * _NP, 128), jnp.float32),
         jnp.zeros((8, 128), jnp.float32)])

    degs = _get_prop(128)(ones_tab, srcp3, deg3, z128)
    dego = degs[0, :_N, 0:1]
    degi = degs[1, :_N, 0:1]
    xs0, nsc, ndc = _prep(h, dego, degi)
    nsp = jnp.pad(nsc, ((0, _NP - _N), (0, 0)))
    ndp = jnp.pad(ndc, ((0, _NP - _N), (0, 0)))
    tab0 = jnp.concatenate([jnp.pad(xs0, ((0, _NP - _N), (0, 0))),
                            jnp.zeros((8, _FLEN), jnp.float32)])

    agg = _get_prop(128, _KPT // 2)(tab0, srcp3_l0, dst3_l0, z128)
    x = _tmat0(agg, ndp, nsp, W0, b0.reshape(1, _H))
    for W, b in ((W1, b1), (W2, b2), (W3, b3), (W4, b4), (W5, b5)):
        agg = _get_prop(128)(_pad_table(x), srcp3, dst3, z128)
        x = _tmat128(agg, ndp, nsp, W, b.reshape(1, _H))
    agg = _get_prop(128)(_pad_table(x), srcp3, dst3, z128)
    return _head(agg, ndp, W6, b6.reshape(1, _H), Wl1, bl1.reshape(1, _H),
                 Wl2, bl2.reshape(1, _IR))
